# baseline (speedup 1.0000x reference)
import functools

import jax
import jax.numpy as jnp
from jax import lax
from jax.experimental import pallas as pl
from jax.experimental.pallas import tpu as pltpu
from jax.experimental.pallas import tpu_sc as plsc

N = 50000
E = 1600000
EPS = 1e-5

NC = 2
NS = 16
NW = NC * NS
PER_W = E // NW
CHUNK = 400
NCHUNK = PER_W // CHUNK
LANES = 16

_mesh = plsc.VectorSubcoreMesh(core_axis_name="c", subcore_axis_name="s")


NPAD = 51200


@functools.partial(
    pl.kernel,
    mesh=_mesh,
    compiler_params=pltpu.CompilerParams(use_tc_tiling_on_sc=False),
    out_type=jax.ShapeDtypeStruct((2, NC, NPAD), jnp.float32),
    scratch_types=[
        pltpu.VMEM((CHUNK,), jnp.int32),
        pltpu.VMEM((CHUNK,), jnp.int32),
        pltpu.VMEM((CHUNK,), jnp.float32),
        pltpu.VMEM((3200,), jnp.float32),
        pltpu.VMEM_SHARED((NPAD,), jnp.float32),
        pltpu.VMEM_SHARED((NPAD,), jnp.float32),
    ],
)
def _sc_degrees(src_hbm, dst_hbm, out_hbm,
                src_v, dst_v, ones_v, zbuf, deg_sh, cnt_sh):
    c = lax.axis_index("c")
    s = lax.axis_index("s")
    wid = c * NS + s

    def fill(i, _):
        zbuf[pl.ds(i * LANES, LANES)] = jnp.zeros((LANES,), jnp.float32)
        return 0

    lax.fori_loop(0, 3200 // LANES, fill, 0)

    def fill1(i, _):
        ones_v[pl.ds(i * LANES, LANES)] = jnp.ones((LANES,), jnp.float32)
        return 0

    lax.fori_loop(0, CHUNK // LANES, fill1, 0)

    pltpu.sync_copy(zbuf, deg_sh.at[pl.ds(s * 3200, 3200)])
    pltpu.sync_copy(zbuf, cnt_sh.at[pl.ds(s * 3200, 3200)])
    plsc.subcore_barrier()

    def chunk(i, _):
        base = wid * PER_W + i * CHUNK
        pltpu.sync_copy(src_hbm.at[pl.ds(base, CHUNK)], src_v)
        pltpu.sync_copy(dst_hbm.at[pl.ds(base, CHUNK)], dst_v)
        pltpu.sync_copy(ones_v, deg_sh.at[src_v], add=True)
        pltpu.sync_copy(ones_v, cnt_sh.at[dst_v], add=True)
        return 0

    lax.fori_loop(0, NCHUNK, chunk, 0)
    plsc.subcore_barrier()

    pltpu.sync_copy(deg_sh.at[pl.ds(s * 3200, 3200)],
                    out_hbm.at[0, c, pl.ds(s * 3200, 3200)])
    pltpu.sync_copy(cnt_sh.at[pl.ds(s * 3200, 3200)],
                    out_hbm.at[1, c, pl.ds(s * 3200, 3200)])


_TROWS = 3200
_ZROWS = 400


def _rne_bf16(v):
    u = lax.bitcast_convert_type(v, jnp.int32)
    u = (u + jnp.int32(0x7FFF) + ((u >> 16) & jnp.int32(1))) & jnp.int32(-65536)
    return lax.bitcast_convert_type(u, jnp.float32)


@functools.partial(
    pl.kernel,
    mesh=_mesh,
    compiler_params=pltpu.CompilerParams(use_tc_tiling_on_sc=False),
    out_type=jax.ShapeDtypeStruct((NC, N, LANES), jnp.float32),
    scratch_types=[
        pltpu.VMEM((CHUNK,), jnp.int32),
        pltpu.VMEM((CHUNK,), jnp.int32),
        pltpu.VMEM((CHUNK,), jnp.int32),
        pltpu.VMEM((CHUNK,), jnp.int32),
        pltpu.VMEM((CHUNK,), jnp.float32),
        pltpu.VMEM((CHUNK,), jnp.float32),
        pltpu.VMEM((CHUNK,), jnp.float32),
        pltpu.VMEM((CHUNK,), jnp.float32),
        pltpu.VMEM((CHUNK, LANES), jnp.float32),
        pltpu.VMEM((CHUNK, LANES), jnp.float32),
        pltpu.VMEM((LANES,), jnp.float32),
        pltpu.VMEM((LANES,), jnp.float32),
        pltpu.VMEM((LANES,), jnp.float32),
        pltpu.VMEM((_ZROWS, LANES), jnp.float32),
        pltpu.VMEM_SHARED((N, LANES), jnp.float32),
        pltpu.SemaphoreType.DMA,
        pltpu.SemaphoreType.DMA,
    ],
)
def _sc_edge(src_hbm, dst_hbm, eac0_hbm, eac1_hbm, g_hbm, a0_hbm, a1_hbm,
             bn_hbm, out_hbm,
             src_a, dst_a, src_b, dst_b, ea0_a, ea1_a, ea0_b, ea1_b,
             rows_a, rows_b, a0_v, a1_v, bn_v, zbuf, acc_sh, sem_a, sem_b):
    c = lax.axis_index("c")
    s = lax.axis_index("s")
    wid = c * NS + s

    def zb(i, _):
        zbuf[i] = jnp.zeros((LANES,), jnp.float32)
        return 0

    lax.fori_loop(0, _ZROWS, zb, 0)
    row0 = s * _TROWS
    for j in range(_TROWS // _ZROWS):
        off = row0 + j * _ZROWS

        @pl.when(off < N)
        def _z():
            pltpu.sync_copy(zbuf, acc_sh.at[pl.ds(off, _ZROWS)])

    plsc.subcore_barrier()

    pltpu.sync_copy(a0_hbm, a0_v)
    pltpu.sync_copy(a1_hbm, a1_v)
    pltpu.sync_copy(bn_hbm, bn_v)
    a0 = a0_v[...]
    a1 = a1_v[...]
    bnv = bn_v[...]
    s0 = bnv[0]
    f0 = bnv[1]
    s1 = bnv[2]
    f1 = bnv[3]
    base0 = wid * PER_W

    def load(ch, sv, dv, e0v, e1v):
        b = base0 + ch * CHUNK
        pltpu.sync_copy(src_hbm.at[pl.ds(b, CHUNK)], sv)
        pltpu.sync_copy(dst_hbm.at[pl.ds(b, CHUNK)], dv)
        pltpu.sync_copy(eac0_hbm.at[pl.ds(b, CHUNK)], e0v)
        pltpu.sync_copy(eac1_hbm.at[pl.ds(b, CHUNK)], e1v)

    def compute(rows_v, e0v, e1v, dv):
        def group(gi, _):
            e0 = gi * LANES
            h0 = _rne_bf16(e0v[pl.ds(e0, LANES)] * s0 + f0)
            h1 = _rne_bf16(e1v[pl.ds(e0, LANES)] * s1 + f1)
            for j in range(LANES):
                e = e0 + j
                t = rows_v[e] + a0 * h0[j] + a1 * h1[j]
                rows_v[e] = jnp.maximum(t, 0.0)
            return 0

        lax.fori_loop(0, CHUNK // LANES, group, 0)
        pltpu.sync_copy(rows_v, acc_sh.at[dv], add=True)

    load(0, src_a, dst_a, ea0_a, ea1_a)
    pltpu.async_copy(g_hbm.at[src_a], rows_a, sem_a)
    load(1, src_b, dst_b, ea0_b, ea1_b)
    pltpu.async_copy(g_hbm.at[src_b], rows_b, sem_b)

    def body(i, _):
        pltpu.make_async_copy(g_hbm.at[src_a], rows_a, sem_a).wait()
        compute(rows_a, ea0_a, ea1_a, dst_a)

        @pl.when(2 * i + 2 < NCHUNK)
        def _na():
            load(2 * i + 2, src_a, dst_a, ea0_a, ea1_a)
            pltpu.async_copy(g_hbm.at[src_a], rows_a, sem_a)

        @pl.when(2 * i + 1 < NCHUNK)
        def _bb():
            pltpu.make_async_copy(g_hbm.at[src_b], rows_b, sem_b).wait()
            compute(rows_b, ea0_b, ea1_b, dst_b)

            @pl.when(2 * i + 3 < NCHUNK)
            def _nb():
                load(2 * i + 3, src_b, dst_b, ea0_b, ea1_b)
                pltpu.async_copy(g_hbm.at[src_b], rows_b, sem_b)

        return 0

    lax.fori_loop(0, (NCHUNK + 1) // 2, body, 0)
    plsc.subcore_barrier()
    for j in range(_TROWS // _ZROWS):
        off = row0 + j * _ZROWS

        @pl.when(off < N)
        def _wb():
            pltpu.sync_copy(acc_sh.at[pl.ds(off, _ZROWS)],
                            out_hbm.at[c, pl.ds(off, _ZROWS)])


def _stats_fold(mean, msq, gamma, beta):
    var = msq - mean * mean
    scale = gamma * lax.rsqrt(var + EPS)
    shift = beta - mean * scale
    return scale, shift


def _r16(x):
    return x.astype(jnp.bfloat16).astype(jnp.float32)


def _pad16(v, axis):
    w = v.shape[axis]
    if w == LANES:
        return v
    pads = list(v.shape)
    pads[axis] = LANES - w
    return jnp.concatenate([v, jnp.zeros(pads, v.dtype)], axis=axis)


def _tc_prep_body(ea0_ref, ea1_ref, xt_ref, degs_ref,
                  g1_ref, b1_ref, w1_ref, bb1_ref,
                  gt_ref, a0_ref, a1_ref, bn_ref, invt_ref, degt_ref, east_ref):
    ea0 = ea0_ref[...]
    ea1 = ea1_ref[...]
    s0 = jnp.sum(ea0)
    q0 = jnp.sum(ea0 * ea0)
    s1 = jnp.sum(ea1)
    q1 = jnp.sum(ea1 * ea1)
    east_ref[...] = jnp.stack([jnp.stack([s0, s1]), jnp.stack([q0, q1])])

    deg = degs_ref[0, 0:1, :N] + degs_ref[0, 1:2, :N]
    cnt = degs_ref[1, 0:1, :N] + degs_ref[1, 1:2, :N]
    degt_ref[...] = deg
    invt_ref[...] = 1.0 / jnp.maximum(cnt, 1.0)

    x = xt_ref[...]
    ef = jnp.float32(E)
    sx = jnp.sum(deg * x)
    qx = jnp.sum(deg * x * x)
    mean = jnp.stack([s0, s1, sx]) / ef
    msq = jnp.stack([q0, q1, qx]) / ef
    sc, sf = _stats_fold(mean, msq, g1_ref[...], b1_ref[...])
    w1 = w1_ref[...]
    hnx = _r16(x * sc[2] + sf[2])
    gt = _r16(w1[:, 2:3]) * hnx + bb1_ref[...][:, None]
    gt_ref[...] = _pad16(gt, 0)
    a0_ref[...] = _pad16(_r16(w1[:, 0]), 0)
    a1_ref[...] = _pad16(_r16(w1[:, 1]), 0)
    bn_ref[...] = _pad16(jnp.stack([sc[0], sf[0], sc[1], sf[1]]), 0)


def _tc_prep(ea0r, ea1r, xt, degs, p1):
    g1, b1, w1, bb1 = p1
    return pl.pallas_call(
        _tc_prep_body,
        out_shape=[
            jax.ShapeDtypeStruct((LANES, N), jnp.float32),
            jax.ShapeDtypeStruct((LANES,), jnp.float32),
            jax.ShapeDtypeStruct((LANES,), jnp.float32),
            jax.ShapeDtypeStruct((LANES,), jnp.float32),
            jax.ShapeDtypeStruct((1, N), jnp.float32),
            jax.ShapeDtypeStruct((1, N), jnp.float32),
            jax.ShapeDtypeStruct((2, 2), jnp.float32),
        ],
    )(ea0r, ea1r, xt, degs, g1, b1, w1, bb1)


def _tc_node_body(wdim, last, s01_ref, invt_ref, hidt_ref, degt_ref, east_ref,
                  g2_ref, b2_ref, w2_ref, bb2_ref,
                  *rest):
    if last:
        (hout_ref,) = rest
    else:
        (g1_ref, b1_ref, w1_ref, bb1_ref,
         hout_ref, gt_ref, a0_ref, a1_ref, bn_ref) = rest

    red = (s01_ref[0] + s01_ref[1])[:wdim] * invt_ref[...]
    z = jnp.concatenate([red, hidt_ref[...]], axis=0)
    nf = jnp.float32(N)
    m = jnp.sum(z, axis=1) / nf
    msq = jnp.sum(z * z, axis=1) / nf
    sc2, sf2 = _stats_fold(m, msq, g2_ref[...], b2_ref[...])
    zbn = (z * sc2[:, None] + sf2[:, None]).astype(jnp.bfloat16)
    h = jnp.maximum(
        jnp.dot(w2_ref[...].astype(jnp.bfloat16), zbn,
                preferred_element_type=jnp.float32)
        + bb2_ref[...][:, None], 0.0)
    hout_ref[...] = h
    if last:
        return

    deg = degt_ref[...]
    ef = jnp.float32(E)
    sh = jnp.sum(deg * h, axis=1)
    qh = jnp.sum(deg * h * h, axis=1)
    east = east_ref[...]
    mean1 = jnp.concatenate([east[0] / ef, sh / ef])
    msq1 = jnp.concatenate([east[1] / ef, qh / ef])
    sc1, sf1 = _stats_fold(mean1, msq1, g1_ref[...], b1_ref[...])
    w1 = w1_ref[...]
    hnh = (h * sc1[2:, None] + sf1[2:, None]).astype(jnp.bfloat16)
    gt = (jnp.dot(w1[:, 2:].astype(jnp.bfloat16), hnh,
                  preferred_element_type=jnp.float32)
          + bb1_ref[...][:, None])
    gt_ref[...] = _pad16(gt, 0)
    a0_ref[...] = _pad16(_r16(w1[:, 0]), 0)
    a1_ref[...] = _pad16(_r16(w1[:, 1]), 0)
    bn_ref[...] = _pad16(jnp.stack([sc1[0], sf1[0], sc1[1], sf1[1]]), 0)


def _tc_node(s01t, invt, hidt, degt, east, p2, p1n, wdim):
    g2, b2, w2, bb2 = p2
    kout = w2.shape[0]
    last = p1n is None
    outs = [jax.ShapeDtypeStruct((kout, N), jnp.float32)]
    args = [s01t, invt, hidt, degt, east, g2, b2, w2, bb2]
    if not last:
        g1, b1, w1, bb1 = p1n
        args = args + [g1, b1, w1, bb1]
        outs += [
            jax.ShapeDtypeStruct((LANES, N), jnp.float32),
            jax.ShapeDtypeStruct((LANES,), jnp.float32),
            jax.ShapeDtypeStruct((LANES,), jnp.float32),
            jax.ShapeDtypeStruct((LANES,), jnp.float32),
        ]
    body = functools.partial(_tc_node_body, wdim, last)
    return pl.pallas_call(body, out_shape=outs)(*args)


def kernel(x, edge_index, edge_attr, params):
    src = edge_index[0]
    dst = edge_index[1]
    ea0 = edge_attr[:, 0]
    ea1 = edge_attr[:, 1]

    degs = _sc_degrees(src, dst)
    p1_0 = params[0][0][0]
    gt, a0, a1, bn01, invt, degt, east = _tc_prep(
        ea0.reshape(E // 128, 128), ea1.reshape(E // 128, 128),
        x.T, degs, p1_0)

    hidt = x.T
    for l in range(4):
        s01 = _sc_edge(src, dst, ea0, ea1, gt.T, a0, a1, bn01)
        s01t = jnp.transpose(s01, (0, 2, 1))
        wdim = params[l][0][0][2].shape[0]
        p2 = params[l][1][0]
        p1n = params[l + 1][0][0] if l < 3 else None
        res = _tc_node(s01t, invt, hidt, degt, east, p2, p1n, wdim)
        if l < 3:
            hidt, gt, a0, a1, bn01 = res
        else:
            hidt = res[0]
    return hidt.T


# speedup vs baseline: 1.3683x; 1.3683x over previous
import functools

import jax
import jax.numpy as jnp
from jax import lax
from jax.experimental import pallas as pl
from jax.experimental.pallas import tpu as pltpu
from jax.experimental.pallas import tpu_sc as plsc

N = 50000
E = 1600000
EPS = 1e-5

NC = 2
NS = 16
NW = NC * NS
PER_W = E // NW
CHUNK = 2000
NCHUNK = PER_W // CHUNK
LANES = 16

_mesh = plsc.VectorSubcoreMesh(core_axis_name="c", subcore_axis_name="s")


NPAD = 51200


@functools.partial(
    pl.kernel,
    mesh=_mesh,
    compiler_params=pltpu.CompilerParams(use_tc_tiling_on_sc=False),
    out_type=jax.ShapeDtypeStruct((2, NC, NPAD), jnp.float32),
    scratch_types=[
        pltpu.VMEM((CHUNK,), jnp.int32),
        pltpu.VMEM((CHUNK,), jnp.int32),
        pltpu.VMEM((CHUNK,), jnp.float32),
        pltpu.VMEM((3200,), jnp.float32),
        pltpu.VMEM_SHARED((NPAD,), jnp.float32),
        pltpu.VMEM_SHARED((NPAD,), jnp.float32),
    ],
)
def _sc_degrees(src_hbm, dst_hbm, out_hbm,
                src_v, dst_v, ones_v, zbuf, deg_sh, cnt_sh):
    c = lax.axis_index("c")
    s = lax.axis_index("s")
    wid = c * NS + s

    def fill(i, _):
        zbuf[pl.ds(i * LANES, LANES)] = jnp.zeros((LANES,), jnp.float32)
        return 0

    lax.fori_loop(0, 3200 // LANES, fill, 0)

    def fill1(i, _):
        ones_v[pl.ds(i * LANES, LANES)] = jnp.ones((LANES,), jnp.float32)
        return 0

    lax.fori_loop(0, CHUNK // LANES, fill1, 0)

    pltpu.sync_copy(zbuf, deg_sh.at[pl.ds(s * 3200, 3200)])
    pltpu.sync_copy(zbuf, cnt_sh.at[pl.ds(s * 3200, 3200)])
    plsc.subcore_barrier()

    def chunk(i, _):
        base = wid * PER_W + i * CHUNK
        pltpu.sync_copy(src_hbm.at[pl.ds(base, CHUNK)], src_v)
        pltpu.sync_copy(dst_hbm.at[pl.ds(base, CHUNK)], dst_v)
        pltpu.sync_copy(ones_v, deg_sh.at[src_v], add=True)
        pltpu.sync_copy(ones_v, cnt_sh.at[dst_v], add=True)
        return 0

    lax.fori_loop(0, NCHUNK, chunk, 0)
    plsc.subcore_barrier()

    pltpu.sync_copy(deg_sh.at[pl.ds(s * 3200, 3200)],
                    out_hbm.at[0, c, pl.ds(s * 3200, 3200)])
    pltpu.sync_copy(cnt_sh.at[pl.ds(s * 3200, 3200)],
                    out_hbm.at[1, c, pl.ds(s * 3200, 3200)])


_TROWS = 3200
_ZROWS = 400


def _rne_bf16(v):
    u = lax.bitcast_convert_type(v, jnp.int32)
    u = (u + jnp.int32(0x7FFF) + ((u >> 16) & jnp.int32(1))) & jnp.int32(-65536)
    return lax.bitcast_convert_type(u, jnp.float32)


@functools.partial(
    pl.kernel,
    mesh=_mesh,
    compiler_params=pltpu.CompilerParams(use_tc_tiling_on_sc=False),
    out_type=jax.ShapeDtypeStruct((NC, N, LANES), jnp.float32),
    scratch_types=[
        pltpu.VMEM((CHUNK,), jnp.int32),
        pltpu.VMEM((CHUNK,), jnp.int32),
        pltpu.VMEM((CHUNK,), jnp.float32),
        pltpu.VMEM((CHUNK,), jnp.float32),
        pltpu.VMEM((CHUNK, LANES), jnp.float32),
        pltpu.VMEM((LANES,), jnp.float32),
        pltpu.VMEM((LANES,), jnp.float32),
        pltpu.VMEM((LANES,), jnp.float32),
        pltpu.VMEM((_ZROWS, LANES), jnp.float32),
        pltpu.VMEM_SHARED((N, LANES), jnp.float32),
        pltpu.SemaphoreType.DMA,
    ],
)
def _sc_edge(src_hbm, dst_hbm, eac0_hbm, eac1_hbm, g_hbm, a0_hbm, a1_hbm,
             bn_hbm, out_hbm,
             src_v, dst_v, ea0_v, ea1_v, rows_v, a0_v, a1_v, bn_v, zbuf,
             acc_sh, sem):
    c = lax.axis_index("c")
    s = lax.axis_index("s")
    wid = c * NS + s

    def zb(i, _):
        zbuf[i] = jnp.zeros((LANES,), jnp.float32)
        return 0

    lax.fori_loop(0, _ZROWS, zb, 0)
    row0 = s * _TROWS
    for j in range(_TROWS // _ZROWS):
        off = row0 + j * _ZROWS

        @pl.when(off < N)
        def _z():
            pltpu.sync_copy(zbuf, acc_sh.at[pl.ds(off, _ZROWS)])

    plsc.subcore_barrier()

    pltpu.sync_copy(a0_hbm, a0_v)
    pltpu.sync_copy(a1_hbm, a1_v)
    pltpu.sync_copy(bn_hbm, bn_v)
    a0 = a0_v[...]
    a1 = a1_v[...]
    bnv = bn_v[...]
    s0 = bnv[0]
    f0 = bnv[1]
    s1 = bnv[2]
    f1 = bnv[3]
    def chunk(i, _):
        base = wid * PER_W + i * CHUNK
        pltpu.sync_copy(src_hbm.at[pl.ds(base, CHUNK)], src_v)
        pltpu.sync_copy(dst_hbm.at[pl.ds(base, CHUNK)], dst_v)
        pltpu.sync_copy(eac0_hbm.at[pl.ds(base, CHUNK)], ea0_v)
        pltpu.sync_copy(eac1_hbm.at[pl.ds(base, CHUNK)], ea1_v)
        pltpu.async_copy(g_hbm.at[src_v], rows_v, sem).wait()

        def group(gi, _):
            e0 = gi * LANES
            h0 = _rne_bf16(ea0_v[pl.ds(e0, LANES)] * s0 + f0)
            h1 = _rne_bf16(ea1_v[pl.ds(e0, LANES)] * s1 + f1)
            for j in range(LANES):
                e = e0 + j
                t = rows_v[e] + a0 * h0[j] + a1 * h1[j]
                rows_v[e] = jnp.maximum(t, 0.0)
            return 0

        lax.fori_loop(0, CHUNK // LANES, group, 0)
        pltpu.sync_copy(rows_v, acc_sh.at[dst_v], add=True)
        return 0

    lax.fori_loop(0, NCHUNK, chunk, 0)
    plsc.subcore_barrier()
    for j in range(_TROWS // _ZROWS):
        off = row0 + j * _ZROWS

        @pl.when(off < N)
        def _wb():
            pltpu.sync_copy(acc_sh.at[pl.ds(off, _ZROWS)],
                            out_hbm.at[c, pl.ds(off, _ZROWS)])


def _stats_fold(mean, msq, gamma, beta):
    var = msq - mean * mean
    scale = gamma * lax.rsqrt(var + EPS)
    shift = beta - mean * scale
    return scale, shift


def _r16(x):
    return x.astype(jnp.bfloat16).astype(jnp.float32)


def _pad16(v, axis):
    w = v.shape[axis]
    if w == LANES:
        return v
    pads = list(v.shape)
    pads[axis] = LANES - w
    return jnp.concatenate([v, jnp.zeros(pads, v.dtype)], axis=axis)


def _tc_prep_body(ea0_ref, ea1_ref, xt_ref, degs_ref,
                  g1_ref, b1_ref, w1_ref, bb1_ref,
                  gt_ref, a0_ref, a1_ref, bn_ref, invt_ref, degt_ref, east_ref):
    ea0 = ea0_ref[...]
    ea1 = ea1_ref[...]
    s0 = jnp.sum(ea0)
    q0 = jnp.sum(ea0 * ea0)
    s1 = jnp.sum(ea1)
    q1 = jnp.sum(ea1 * ea1)
    east_ref[...] = jnp.stack([jnp.stack([s0, s1]), jnp.stack([q0, q1])])

    deg = degs_ref[0, 0:1, :N] + degs_ref[0, 1:2, :N]
    cnt = degs_ref[1, 0:1, :N] + degs_ref[1, 1:2, :N]
    degt_ref[...] = deg
    invt_ref[...] = 1.0 / jnp.maximum(cnt, 1.0)

    x = xt_ref[...]
    ef = jnp.float32(E)
    sx = jnp.sum(deg * x)
    qx = jnp.sum(deg * x * x)
    mean = jnp.stack([s0, s1, sx]) / ef
    msq = jnp.stack([q0, q1, qx]) / ef
    sc, sf = _stats_fold(mean, msq, g1_ref[...], b1_ref[...])
    w1 = w1_ref[...]
    hnx = _r16(x * sc[2] + sf[2])
    gt = _r16(w1[:, 2:3]) * hnx + bb1_ref[...][:, None]
    gt_ref[...] = _pad16(gt, 0)
    a0_ref[...] = _pad16(_r16(w1[:, 0]), 0)
    a1_ref[...] = _pad16(_r16(w1[:, 1]), 0)
    bn_ref[...] = _pad16(jnp.stack([sc[0], sf[0], sc[1], sf[1]]), 0)


def _tc_prep(ea0r, ea1r, xt, degs, p1):
    g1, b1, w1, bb1 = p1
    return pl.pallas_call(
        _tc_prep_body,
        out_shape=[
            jax.ShapeDtypeStruct((LANES, N), jnp.float32),
            jax.ShapeDtypeStruct((LANES,), jnp.float32),
            jax.ShapeDtypeStruct((LANES,), jnp.float32),
            jax.ShapeDtypeStruct((LANES,), jnp.float32),
            jax.ShapeDtypeStruct((1, N), jnp.float32),
            jax.ShapeDtypeStruct((1, N), jnp.float32),
            jax.ShapeDtypeStruct((2, 2), jnp.float32),
        ],
    )(ea0r, ea1r, xt, degs, g1, b1, w1, bb1)


def _tc_node_body(wdim, last, s01_ref, invt_ref, hidt_ref, degt_ref, east_ref,
                  g2_ref, b2_ref, w2_ref, bb2_ref,
                  *rest):
    if last:
        (hout_ref,) = rest
    else:
        (g1_ref, b1_ref, w1_ref, bb1_ref,
         hout_ref, gt_ref, a0_ref, a1_ref, bn_ref) = rest

    red = (s01_ref[0] + s01_ref[1])[:wdim] * invt_ref[...]
    z = jnp.concatenate([red, hidt_ref[...]], axis=0)
    nf = jnp.float32(N)
    m = jnp.sum(z, axis=1) / nf
    msq = jnp.sum(z * z, axis=1) / nf
    sc2, sf2 = _stats_fold(m, msq, g2_ref[...], b2_ref[...])
    zbn = (z * sc2[:, None] + sf2[:, None]).astype(jnp.bfloat16)
    h = jnp.maximum(
        jnp.dot(w2_ref[...].astype(jnp.bfloat16), zbn,
                preferred_element_type=jnp.float32)
        + bb2_ref[...][:, None], 0.0)
    hout_ref[...] = h
    if last:
        return

    deg = degt_ref[...]
    ef = jnp.float32(E)
    sh = jnp.sum(deg * h, axis=1)
    qh = jnp.sum(deg * h * h, axis=1)
    east = east_ref[...]
    mean1 = jnp.concatenate([east[0] / ef, sh / ef])
    msq1 = jnp.concatenate([east[1] / ef, qh / ef])
    sc1, sf1 = _stats_fold(mean1, msq1, g1_ref[...], b1_ref[...])
    w1 = w1_ref[...]
    hnh = (h * sc1[2:, None] + sf1[2:, None]).astype(jnp.bfloat16)
    gt = (jnp.dot(w1[:, 2:].astype(jnp.bfloat16), hnh,
                  preferred_element_type=jnp.float32)
          + bb1_ref[...][:, None])
    gt_ref[...] = _pad16(gt, 0)
    a0_ref[...] = _pad16(_r16(w1[:, 0]), 0)
    a1_ref[...] = _pad16(_r16(w1[:, 1]), 0)
    bn_ref[...] = _pad16(jnp.stack([sc1[0], sf1[0], sc1[1], sf1[1]]), 0)


def _tc_node(s01t, invt, hidt, degt, east, p2, p1n, wdim):
    g2, b2, w2, bb2 = p2
    kout = w2.shape[0]
    last = p1n is None
    outs = [jax.ShapeDtypeStruct((kout, N), jnp.float32)]
    args = [s01t, invt, hidt, degt, east, g2, b2, w2, bb2]
    if not last:
        g1, b1, w1, bb1 = p1n
        args = args + [g1, b1, w1, bb1]
        outs += [
            jax.ShapeDtypeStruct((LANES, N), jnp.float32),
            jax.ShapeDtypeStruct((LANES,), jnp.float32),
            jax.ShapeDtypeStruct((LANES,), jnp.float32),
            jax.ShapeDtypeStruct((LANES,), jnp.float32),
        ]
    body = functools.partial(_tc_node_body, wdim, last)
    return pl.pallas_call(body, out_shape=outs)(*args)


def kernel(x, edge_index, edge_attr, params):
    src = edge_index[0]
    dst = edge_index[1]
    ea0 = edge_attr[:, 0]
    ea1 = edge_attr[:, 1]

    degs = _sc_degrees(src, dst)
    p1_0 = params[0][0][0]
    gt, a0, a1, bn01, invt, degt, east = _tc_prep(
        ea0.reshape(E // 128, 128), ea1.reshape(E // 128, 128),
        x.T, degs, p1_0)

    hidt = x.T
    for l in range(4):
        s01 = _sc_edge(src, dst, ea0, ea1, gt.T, a0, a1, bn01)
        s01t = jnp.transpose(s01, (0, 2, 1))
        wdim = params[l][0][0][2].shape[0]
        p2 = params[l][1][0]
        p1n = params[l + 1][0][0] if l < 3 else None
        res = _tc_node(s01t, invt, hidt, degt, east, p2, p1n, wdim)
        if l < 3:
            hidt, gt, a0, a1, bn01 = res
        else:
            hidt = res[0]
    return hidt.T


# speedup vs baseline: 1.5560x; 1.1372x over previous
import functools

import jax
import jax.numpy as jnp
from jax import lax
from jax.experimental import pallas as pl
from jax.experimental.pallas import tpu as pltpu
from jax.experimental.pallas import tpu_sc as plsc

N = 50000
E = 1600000
EPS = 1e-5

NC = 2
NS = 16
NW = NC * NS
PER_W = E // NW
CHUNK = 2000
NCHUNK = PER_W // CHUNK
LANES = 16

_mesh = plsc.VectorSubcoreMesh(core_axis_name="c", subcore_axis_name="s")


NPAD = 51200


@functools.partial(
    pl.kernel,
    mesh=_mesh,
    compiler_params=pltpu.CompilerParams(use_tc_tiling_on_sc=False),
    out_type=jax.ShapeDtypeStruct((2, NC, NPAD), jnp.float32),
    scratch_types=[
        pltpu.VMEM((CHUNK,), jnp.int32),
        pltpu.VMEM((CHUNK,), jnp.int32),
        pltpu.VMEM((CHUNK,), jnp.float32),
        pltpu.VMEM((3200,), jnp.float32),
        pltpu.VMEM_SHARED((NPAD,), jnp.float32),
        pltpu.VMEM_SHARED((NPAD,), jnp.float32),
    ],
)
def _sc_degrees(src_hbm, dst_hbm, out_hbm,
                src_v, dst_v, ones_v, zbuf, deg_sh, cnt_sh):
    c = lax.axis_index("c")
    s = lax.axis_index("s")
    wid = c * NS + s

    def fill(i, _):
        zbuf[pl.ds(i * LANES, LANES)] = jnp.zeros((LANES,), jnp.float32)
        return 0

    lax.fori_loop(0, 3200 // LANES, fill, 0)

    def fill1(i, _):
        ones_v[pl.ds(i * LANES, LANES)] = jnp.ones((LANES,), jnp.float32)
        return 0

    lax.fori_loop(0, CHUNK // LANES, fill1, 0)

    pltpu.sync_copy(zbuf, deg_sh.at[pl.ds(s * 3200, 3200)])
    pltpu.sync_copy(zbuf, cnt_sh.at[pl.ds(s * 3200, 3200)])
    plsc.subcore_barrier()

    def chunk(i, _):
        base = wid * PER_W + i * CHUNK
        pltpu.sync_copy(src_hbm.at[pl.ds(base, CHUNK)], src_v)
        pltpu.sync_copy(dst_hbm.at[pl.ds(base, CHUNK)], dst_v)
        pltpu.sync_copy(ones_v, deg_sh.at[src_v], add=True)
        pltpu.sync_copy(ones_v, cnt_sh.at[dst_v], add=True)
        return 0

    lax.fori_loop(0, NCHUNK, chunk, 0)
    plsc.subcore_barrier()

    pltpu.sync_copy(deg_sh.at[pl.ds(s * 3200, 3200)],
                    out_hbm.at[0, c, pl.ds(s * 3200, 3200)])
    pltpu.sync_copy(cnt_sh.at[pl.ds(s * 3200, 3200)],
                    out_hbm.at[1, c, pl.ds(s * 3200, 3200)])


_TROWS = 3200
_ZROWS = 100


def _rne_bf16(v):
    u = lax.bitcast_convert_type(v, jnp.int32)
    u = (u + jnp.int32(0x7FFF) + ((u >> 16) & jnp.int32(1))) & jnp.int32(-65536)
    return lax.bitcast_convert_type(u, jnp.float32)


@functools.partial(
    pl.kernel,
    mesh=_mesh,
    compiler_params=pltpu.CompilerParams(use_tc_tiling_on_sc=False),
    out_type=jax.ShapeDtypeStruct((NC, N, LANES), jnp.float32),
    scratch_types=[
        pltpu.VMEM((CHUNK,), jnp.int32),
        pltpu.VMEM((CHUNK,), jnp.int32),
        pltpu.VMEM((CHUNK,), jnp.int32),
        pltpu.VMEM((CHUNK,), jnp.float32),
        pltpu.VMEM((CHUNK,), jnp.float32),
        pltpu.VMEM((CHUNK, LANES), jnp.float32),
        pltpu.VMEM((CHUNK, LANES), jnp.float32),
        pltpu.VMEM((LANES,), jnp.float32),
        pltpu.VMEM((LANES,), jnp.float32),
        pltpu.VMEM((LANES,), jnp.float32),
        pltpu.VMEM((_ZROWS, LANES), jnp.float32),
        pltpu.VMEM_SHARED((N, LANES), jnp.float32),
        pltpu.SemaphoreType.DMA,
        pltpu.SemaphoreType.DMA,
    ],
)
def _sc_edge(src_hbm, dst_hbm, eac0_hbm, eac1_hbm, g_hbm, a0_hbm, a1_hbm,
             bn_hbm, out_hbm,
             src_a, src_b, dst_v, ea0_v, ea1_v, rows_a, rows_b,
             a0_v, a1_v, bn_v, zbuf, acc_sh, sem_a, sem_b):
    c = lax.axis_index("c")
    s = lax.axis_index("s")
    wid = c * NS + s

    def zb(i, _):
        zbuf[i] = jnp.zeros((LANES,), jnp.float32)
        return 0

    lax.fori_loop(0, _ZROWS, zb, 0)
    row0 = s * _TROWS
    for j in range(_TROWS // _ZROWS):
        off = row0 + j * _ZROWS

        @pl.when(off < N)
        def _z():
            pltpu.sync_copy(zbuf, acc_sh.at[pl.ds(off, _ZROWS)])

    plsc.subcore_barrier()

    pltpu.sync_copy(a0_hbm, a0_v)
    pltpu.sync_copy(a1_hbm, a1_v)
    pltpu.sync_copy(bn_hbm, bn_v)
    a0 = a0_v[...]
    a1 = a1_v[...]
    bnv = bn_v[...]
    s0 = bnv[0]
    f0 = bnv[1]
    s1 = bnv[2]
    f1 = bnv[3]
    base0 = wid * PER_W

    def consume(ch, rows_v):
        base = base0 + ch * CHUNK
        pltpu.sync_copy(dst_hbm.at[pl.ds(base, CHUNK)], dst_v)
        pltpu.sync_copy(eac0_hbm.at[pl.ds(base, CHUNK)], ea0_v)
        pltpu.sync_copy(eac1_hbm.at[pl.ds(base, CHUNK)], ea1_v)

        def group(gi, _):
            e0 = gi * LANES
            h0 = _rne_bf16(ea0_v[pl.ds(e0, LANES)] * s0 + f0)
            h1 = _rne_bf16(ea1_v[pl.ds(e0, LANES)] * s1 + f1)
            for j in range(LANES):
                e = e0 + j
                t = rows_v[e] + a0 * h0[j] + a1 * h1[j]
                rows_v[e] = jnp.maximum(t, 0.0)
            return 0

        lax.fori_loop(0, CHUNK // LANES, group, 0)
        pltpu.sync_copy(rows_v, acc_sh.at[dst_v], add=True)

    def prefetch(ch, src_v, rows_v, sem):
        pltpu.sync_copy(src_hbm.at[pl.ds(base0 + ch * CHUNK, CHUNK)], src_v)
        pltpu.async_copy(g_hbm.at[src_v], rows_v, sem)

    prefetch(0, src_a, rows_a, sem_a)

    def body(i, _):
        ca = 2 * i

        @pl.when(ca + 1 < NCHUNK)
        def _pb():
            prefetch(ca + 1, src_b, rows_b, sem_b)

        pltpu.make_async_copy(g_hbm.at[src_a], rows_a, sem_a).wait()
        consume(ca, rows_a)

        @pl.when(ca + 1 < NCHUNK)
        def _cb():
            @pl.when(ca + 2 < NCHUNK)
            def _pa():
                prefetch(ca + 2, src_a, rows_a, sem_a)

            pltpu.make_async_copy(g_hbm.at[src_b], rows_b, sem_b).wait()
            consume(ca + 1, rows_b)

        return 0

    lax.fori_loop(0, (NCHUNK + 1) // 2, body, 0)
    plsc.subcore_barrier()
    for j in range(_TROWS // _ZROWS):
        off = row0 + j * _ZROWS

        @pl.when(off < N)
        def _wb():
            pltpu.sync_copy(acc_sh.at[pl.ds(off, _ZROWS)],
                            out_hbm.at[c, pl.ds(off, _ZROWS)])


def _stats_fold(mean, msq, gamma, beta):
    var = msq - mean * mean
    scale = gamma * lax.rsqrt(var + EPS)
    shift = beta - mean * scale
    return scale, shift


def _r16(x):
    return x.astype(jnp.bfloat16).astype(jnp.float32)


def _pad16(v, axis):
    w = v.shape[axis]
    if w == LANES:
        return v
    pads = list(v.shape)
    pads[axis] = LANES - w
    return jnp.concatenate([v, jnp.zeros(pads, v.dtype)], axis=axis)


def _tc_prep_body(ea0_ref, ea1_ref, xt_ref, degs_ref,
                  g1_ref, b1_ref, w1_ref, bb1_ref,
                  gt_ref, a0_ref, a1_ref, bn_ref, invt_ref, degt_ref, east_ref):
    ea0 = ea0_ref[...]
    ea1 = ea1_ref[...]
    s0 = jnp.sum(ea0)
    q0 = jnp.sum(ea0 * ea0)
    s1 = jnp.sum(ea1)
    q1 = jnp.sum(ea1 * ea1)
    east_ref[...] = jnp.stack([jnp.stack([s0, s1]), jnp.stack([q0, q1])])

    deg = degs_ref[0, 0:1, :N] + degs_ref[0, 1:2, :N]
    cnt = degs_ref[1, 0:1, :N] + degs_ref[1, 1:2, :N]
    degt_ref[...] = deg
    invt_ref[...] = 1.0 / jnp.maximum(cnt, 1.0)

    x = xt_ref[...]
    ef = jnp.float32(E)
    sx = jnp.sum(deg * x)
    qx = jnp.sum(deg * x * x)
    mean = jnp.stack([s0, s1, sx]) / ef
    msq = jnp.stack([q0, q1, qx]) / ef
    sc, sf = _stats_fold(mean, msq, g1_ref[...], b1_ref[...])
    w1 = w1_ref[...]
    hnx = _r16(x * sc[2] + sf[2])
    gt = _r16(w1[:, 2:3]) * hnx + bb1_ref[...][:, None]
    gt_ref[...] = _pad16(gt, 0)
    a0_ref[...] = _pad16(_r16(w1[:, 0]), 0)
    a1_ref[...] = _pad16(_r16(w1[:, 1]), 0)
    bn_ref[...] = _pad16(jnp.stack([sc[0], sf[0], sc[1], sf[1]]), 0)


def _tc_prep(ea0r, ea1r, xt, degs, p1):
    g1, b1, w1, bb1 = p1
    return pl.pallas_call(
        _tc_prep_body,
        out_shape=[
            jax.ShapeDtypeStruct((LANES, N), jnp.float32),
            jax.ShapeDtypeStruct((LANES,), jnp.float32),
            jax.ShapeDtypeStruct((LANES,), jnp.float32),
            jax.ShapeDtypeStruct((LANES,), jnp.float32),
            jax.ShapeDtypeStruct((1, N), jnp.float32),
            jax.ShapeDtypeStruct((1, N), jnp.float32),
            jax.ShapeDtypeStruct((2, 2), jnp.float32),
        ],
    )(ea0r, ea1r, xt, degs, g1, b1, w1, bb1)


def _tc_node_body(wdim, last, s01_ref, invt_ref, hidt_ref, degt_ref, east_ref,
                  g2_ref, b2_ref, w2_ref, bb2_ref,
                  *rest):
    if last:
        (hout_ref,) = rest
    else:
        (g1_ref, b1_ref, w1_ref, bb1_ref,
         hout_ref, gt_ref, a0_ref, a1_ref, bn_ref) = rest

    red = (s01_ref[0] + s01_ref[1])[:wdim] * invt_ref[...]
    z = jnp.concatenate([red, hidt_ref[...]], axis=0)
    nf = jnp.float32(N)
    m = jnp.sum(z, axis=1) / nf
    msq = jnp.sum(z * z, axis=1) / nf
    sc2, sf2 = _stats_fold(m, msq, g2_ref[...], b2_ref[...])
    zbn = (z * sc2[:, None] + sf2[:, None]).astype(jnp.bfloat16)
    h = jnp.maximum(
        jnp.dot(w2_ref[...].astype(jnp.bfloat16), zbn,
                preferred_element_type=jnp.float32)
        + bb2_ref[...][:, None], 0.0)
    hout_ref[...] = h
    if last:
        return

    deg = degt_ref[...]
    ef = jnp.float32(E)
    sh = jnp.sum(deg * h, axis=1)
    qh = jnp.sum(deg * h * h, axis=1)
    east = east_ref[...]
    mean1 = jnp.concatenate([east[0] / ef, sh / ef])
    msq1 = jnp.concatenate([east[1] / ef, qh / ef])
    sc1, sf1 = _stats_fold(mean1, msq1, g1_ref[...], b1_ref[...])
    w1 = w1_ref[...]
    hnh = (h * sc1[2:, None] + sf1[2:, None]).astype(jnp.bfloat16)
    gt = (jnp.dot(w1[:, 2:].astype(jnp.bfloat16), hnh,
                  preferred_element_type=jnp.float32)
          + bb1_ref[...][:, None])
    gt_ref[...] = _pad16(gt, 0)
    a0_ref[...] = _pad16(_r16(w1[:, 0]), 0)
    a1_ref[...] = _pad16(_r16(w1[:, 1]), 0)
    bn_ref[...] = _pad16(jnp.stack([sc1[0], sf1[0], sc1[1], sf1[1]]), 0)


def _tc_node(s01t, invt, hidt, degt, east, p2, p1n, wdim):
    g2, b2, w2, bb2 = p2
    kout = w2.shape[0]
    last = p1n is None
    outs = [jax.ShapeDtypeStruct((kout, N), jnp.float32)]
    args = [s01t, invt, hidt, degt, east, g2, b2, w2, bb2]
    if not last:
        g1, b1, w1, bb1 = p1n
        args = args + [g1, b1, w1, bb1]
        outs += [
            jax.ShapeDtypeStruct((LANES, N), jnp.float32),
            jax.ShapeDtypeStruct((LANES,), jnp.float32),
            jax.ShapeDtypeStruct((LANES,), jnp.float32),
            jax.ShapeDtypeStruct((LANES,), jnp.float32),
        ]
    body = functools.partial(_tc_node_body, wdim, last)
    return pl.pallas_call(body, out_shape=outs)(*args)


def kernel(x, edge_index, edge_attr, params):
    src = edge_index[0]
    dst = edge_index[1]
    ea0 = edge_attr[:, 0]
    ea1 = edge_attr[:, 1]

    degs = _sc_degrees(src, dst)
    p1_0 = params[0][0][0]
    gt, a0, a1, bn01, invt, degt, east = _tc_prep(
        ea0.reshape(E // 128, 128), ea1.reshape(E // 128, 128),
        x.T, degs, p1_0)

    hidt = x.T
    for l in range(4):
        s01 = _sc_edge(src, dst, ea0, ea1, gt.T, a0, a1, bn01)
        s01t = jnp.transpose(s01, (0, 2, 1))
        wdim = params[l][0][0][2].shape[0]
        p2 = params[l][1][0]
        p1n = params[l + 1][0][0] if l < 3 else None
        res = _tc_node(s01t, invt, hidt, degt, east, p2, p1n, wdim)
        if l < 3:
            hidt, gt, a0, a1, bn01 = res
        else:
            hidt = res[0]
    return hidt.T
